# baseline (speedup 1.0000x reference)
import functools

import jax
import jax.numpy as jnp
from jax import lax
from jax.experimental import pallas as pl
from jax.experimental.pallas import tpu as pltpu
from jax.experimental.pallas import tpu_sc as plsc

_VOCAB = 1000
_VP = 1024
_D = 64
_B, _T = 4, 4096
_N = _B * _T
_NC = 2
_NS = 16
_NW = _NC * _NS
_TPT = _N // _NW
_WPB = _T // _TPT
_CH = 128
_NCH = _TPT // _CH
_RPT = _VP // _NS


def _phase1_body(x_hbm, tok_hbm, out_a, out_cnt,
                 idx_v, xv, xca, xcb, sca, scb, ones_v, z64, z16,
                 acc_sum, acc_sq, acc_cnt,
                 sem_in, sem_z, sem_s, sem_s2, sem_out):
    c = lax.axis_index("c")
    s = lax.axis_index("s")
    w = s * _NC + c

    ld_idx = pltpu.async_copy(tok_hbm.at[w], idx_v, sem_in)
    ld_x = pltpu.async_copy(x_hbm.at[:, :, w], xv, sem_in)

    iota = lax.iota(jnp.int32, 16)
    fr128 = lax.shift_left(lax.bitwise_and(iota, 7), 7)
    fb_gs = [lax.shift_right_logical(iota, 3) + 2 * g for g in range(4)]

    zeros = jnp.zeros((16,), jnp.float32)
    ones = jnp.ones((16,), jnp.float32)

    def zrow(r, carry):
        for j in range(4):
            z64[r, pl.ds(j * 16, 16)] = zeros
        z16[r] = zeros
        return carry
    lax.fori_loop(0, _RPT, zrow, 0)

    def onesrow(r, carry):
        ones_v[r] = ones
        return carry
    lax.fori_loop(0, _CH, onesrow, 0)

    rows = pl.ds(s * _RPT, _RPT)
    z1 = pltpu.async_copy(z64, acc_sum.at[rows], sem_z)
    z2 = pltpu.async_copy(z64, acc_sq.at[rows], sem_z)
    z3 = pltpu.async_copy(z16, acc_cnt.at[rows], sem_z)

    z1.wait()
    z2.wait()
    z3.wait()
    plsc.subcore_barrier()

    ld_idx.wait()
    ld_x.wait()

    xbufs = [xca, xcb]
    sbufs = [sca, scb]
    sems = [sem_s, sem_s2]
    pend = {}
    ones_cps = []
    for ch in range(_NCH):
        p = ch % 2
        if ch >= 2:
            pend[ch - 2][0].wait()
            pend[ch - 2][1].wait()
        xb, sb = xbufs[p], sbufs[p]
        tbl_v = jnp.zeros((16,), jnp.int32) + ch

        @plsc.parallel_loop(0, _CH // 4, 1, unroll=4)
        def transpose_rows(r):
            for k in range(4):
                tl = r * 4 + k
                col_v = fr128 + tl
                for g in range(4):
                    v = plsc.load_gather(xv, [tbl_v, fb_gs[g], col_v])
                    xb[tl, pl.ds(g * 16, 16)] = v
                    sb[tl, pl.ds(g * 16, 16)] = v * v

        idx = idx_v.at[ch]
        pend[ch] = (
            pltpu.async_copy(xb, acc_sum.at[idx], sems[p], add=True),
            pltpu.async_copy(sb, acc_sq.at[idx], sems[p], add=True),
        )
        ones_cps.append(
            pltpu.async_copy(ones_v, acc_cnt.at[idx], sem_z, add=True))
    for ch in (_NCH - 2, _NCH - 1):
        pend[ch][0].wait()
        pend[ch][1].wait()
    for cp in ones_cps:
        cp.wait()

    plsc.subcore_barrier()

    o1 = pltpu.async_copy(acc_sum.at[rows], out_a.at[c, rows, pl.ds(0, _D)],
                          sem_out)
    o2 = pltpu.async_copy(acc_sq.at[rows], out_a.at[c, rows, pl.ds(_D, _D)],
                          sem_out)
    o3 = pltpu.async_copy(acc_cnt.at[rows], out_cnt.at[c, rows], sem_out)
    o1.wait()
    o2.wait()
    o3.wait()


_phase1 = functools.partial(
    pl.kernel,
    out_type=(
        jax.ShapeDtypeStruct((_NC, _VP, 2 * _D), jnp.float32),
        jax.ShapeDtypeStruct((_NC, _VP, 16), jnp.float32),
    ),
    mesh=plsc.VectorSubcoreMesh(
        core_axis_name="c", subcore_axis_name="s",
        num_cores=_NC, num_subcores=_NS),
    scratch_types=[
        pltpu.VMEM((_NCH, _CH), jnp.int32),
        pltpu.VMEM((_B, _D // 8, 1024), jnp.float32),
        pltpu.VMEM((_CH, _D), jnp.float32),
        pltpu.VMEM((_CH, _D), jnp.float32),
        pltpu.VMEM((_CH, _D), jnp.float32),
        pltpu.VMEM((_CH, _D), jnp.float32),
        pltpu.VMEM((_CH, 16), jnp.float32),
        pltpu.VMEM((_RPT, _D), jnp.float32),
        pltpu.VMEM((_RPT, 16), jnp.float32),
        pltpu.VMEM_SHARED((_VP, _D), jnp.float32),
        pltpu.VMEM_SHARED((_VP, _D), jnp.float32),
        pltpu.VMEM_SHARED((_VP, 16), jnp.float32),
        pltpu.SemaphoreType.DMA,
        pltpu.SemaphoreType.DMA,
        pltpu.SemaphoreType.DMA,
        pltpu.SemaphoreType.DMA,
        pltpu.SemaphoreType.DMA,
    ],
    compiler_params=pltpu.CompilerParams(
        use_tc_tiling_on_sc=False, needs_layout_passes=False,
        disable_bounds_checks=True),
)(_phase1_body)


def _finalize_body(a_ref, cnt_ref, loss_ref, nrep_ref):
    a = a_ref[0] + a_ref[1]
    cnt = cnt_ref[0] + cnt_ref[1]
    c = cnt[:, 0:1]
    cm = jnp.maximum(c, 1.0)
    lane = lax.broadcasted_iota(jnp.int32, (_VP, 2 * _D), 1)
    contrib = jnp.where(lane >= _D, a, -(a * a) / cm)
    ss_sum = jnp.sum(contrib, axis=1, keepdims=True)
    var_mean = ss_sum / (jnp.maximum(c - 1.0, 1.0) * _D)
    repeated = c >= 2.0
    nrep = jnp.sum(repeated.astype(jnp.float32))
    total = jnp.sum(jnp.where(repeated, var_mean, 0.0))
    avg = total / jnp.maximum(nrep, 1.0)
    loss = jnp.maximum(1.0 - avg, 0.0)
    loss = jnp.where(nrep > 0.0, loss, 0.0)
    loss_ref[0, 0] = loss
    nrep_ref[0, 0] = nrep.astype(jnp.int32)


_finalize = pl.pallas_call(
    _finalize_body,
    out_shape=(
        jax.ShapeDtypeStruct((1, 1), jnp.float32),
        jax.ShapeDtypeStruct((1, 1), jnp.int32),
    ),
    out_specs=(
        pl.BlockSpec(memory_space=pltpu.SMEM),
        pl.BlockSpec(memory_space=pltpu.SMEM),
    ),
)


@jax.jit
def kernel(semantic_state, token_ids):
    x5 = jnp.transpose(
        semantic_state.reshape(_B, _T // 128, 128, _D // 8, 8),
        (0, 3, 1, 4, 2))
    x6 = x5.reshape(_B, _D // 8, _T // 128, 8 * 128)
    tok = jnp.transpose(
        token_ids.astype(jnp.int32).reshape(_B, _NW, _CH), (1, 0, 2))
    pa, pcnt = _phase1(x6, tok)
    loss, nrep = _finalize(pa, pcnt)
    return loss[0, 0], nrep[0, 0]


# speedup vs baseline: 1.0580x; 1.0580x over previous
import functools

import jax
import jax.numpy as jnp
from jax import lax
from jax.experimental import pallas as pl
from jax.experimental.pallas import tpu as pltpu
from jax.experimental.pallas import tpu_sc as plsc

_VOCAB = 1000
_VP = 1024
_D = 64
_B, _T = 4, 4096
_N = _B * _T
_NC = 2
_NS = 16
_NW = _NC * _NS
_TPT = _N // _NW
_WPB = _T // _TPT
_CH = 128
_NCH = _TPT // _CH
_RPT = _VP // _NS


def _phase1_body(x_hbm, tok_hbm, out_a, out_cnt,
                 idx_v, xv, xca, xcb, sca, scb, ones_v, z64, z16,
                 cnt_v, cnt8,
                 acc_sum, acc_sq, acc_cnt,
                 sem_in, sem_z, sem_s, sem_s2, sem_out):
    c = lax.axis_index("c")
    s = lax.axis_index("s")
    w = s * _NC + c

    ld_idx = pltpu.async_copy(tok_hbm.at[w], idx_v, sem_in)
    ld_x = pltpu.async_copy(x_hbm.at[:, :, w], xv, sem_in)

    iota = lax.iota(jnp.int32, 16)
    fr128 = lax.shift_left(lax.bitwise_and(iota, 7), 7)
    fb_gs = [lax.shift_right_logical(iota, 3) + 2 * g for g in range(4)]

    zeros = jnp.zeros((16,), jnp.float32)
    ones = jnp.ones((16,), jnp.float32)

    def zrow(r, carry):
        for j in range(4):
            z64[r, pl.ds(j * 16, 16)] = zeros
        z16[r] = zeros
        return carry
    lax.fori_loop(0, _RPT, zrow, 0)

    def onesrow(r, carry):
        ones_v[r] = ones
        return carry
    lax.fori_loop(0, _CH, onesrow, 0)

    rows = pl.ds(s * _RPT, _RPT)
    z1 = pltpu.async_copy(z64, acc_sum.at[rows], sem_z)
    z2 = pltpu.async_copy(z64, acc_sq.at[rows], sem_z)
    z3 = pltpu.async_copy(z16, acc_cnt.at[rows], sem_z)

    z1.wait()
    z2.wait()
    z3.wait()
    plsc.subcore_barrier()

    ld_idx.wait()
    ld_x.wait()

    xbufs = [xca, xcb]
    sbufs = [sca, scb]
    sems = [sem_s, sem_s2]
    pend = {}
    ones_cps = []
    for ch in range(_NCH):
        p = ch % 2
        if ch >= 2:
            pend[ch - 2][0].wait()
            pend[ch - 2][1].wait()
        xb, sb = xbufs[p], sbufs[p]
        tbl_v = jnp.zeros((16,), jnp.int32) + ch

        @plsc.parallel_loop(0, _CH // 4, 1, unroll=4)
        def transpose_rows(r):
            for k in range(4):
                tl = r * 4 + k
                col_v = fr128 + tl
                for g in range(4):
                    v = plsc.load_gather(xv, [tbl_v, fb_gs[g], col_v])
                    xb[tl, pl.ds(g * 16, 16)] = v
                    sb[tl, pl.ds(g * 16, 16)] = v * v

        idx = idx_v.at[ch]
        pend[ch] = (
            pltpu.async_copy(xb, acc_sum.at[idx], sems[p], add=True),
            pltpu.async_copy(sb, acc_sq.at[idx], sems[p], add=True),
        )
        ones_cps.append(
            pltpu.async_copy(ones_v, acc_cnt.at[idx], sem_z, add=True))
    for ch in (_NCH - 2, _NCH - 1):
        pend[ch][0].wait()
        pend[ch][1].wait()
    for cp in ones_cps:
        cp.wait()

    plsc.subcore_barrier()

    o1 = pltpu.async_copy(acc_sum.at[rows], out_a.at[c, rows, pl.ds(0, _D)],
                          sem_out)
    o2 = pltpu.async_copy(acc_sq.at[rows], out_a.at[c, rows, pl.ds(_D, _D)],
                          sem_out)
    pltpu.sync_copy(acc_cnt.at[rows], cnt_v)

    def cntrow(r, carry):
        q = r // 8
        j = (r % 8) * 16
        cnt8[q, pl.ds(j, 16)] = cnt_v[r]
        return carry
    lax.fori_loop(0, _RPT, cntrow, 0)
    o3 = pltpu.async_copy(cnt8, out_cnt.at[c, pl.ds(s * (_RPT // 8), _RPT // 8)],
                          sem_out)
    o1.wait()
    o2.wait()
    o3.wait()


_phase1 = functools.partial(
    pl.kernel,
    out_type=(
        jax.ShapeDtypeStruct((_NC, _VP, 2 * _D), jnp.float32),
        jax.ShapeDtypeStruct((_NC, _VP // 8, 2 * _D), jnp.float32),
    ),
    mesh=plsc.VectorSubcoreMesh(
        core_axis_name="c", subcore_axis_name="s",
        num_cores=_NC, num_subcores=_NS),
    scratch_types=[
        pltpu.VMEM((_NCH, _CH), jnp.int32),
        pltpu.VMEM((_B, _D // 8, 1024), jnp.float32),
        pltpu.VMEM((_CH, _D), jnp.float32),
        pltpu.VMEM((_CH, _D), jnp.float32),
        pltpu.VMEM((_CH, _D), jnp.float32),
        pltpu.VMEM((_CH, _D), jnp.float32),
        pltpu.VMEM((_CH, 16), jnp.float32),
        pltpu.VMEM((_RPT, _D), jnp.float32),
        pltpu.VMEM((_RPT, 16), jnp.float32),
        pltpu.VMEM((_RPT, 16), jnp.float32),
        pltpu.VMEM((_RPT // 8, 2 * _D), jnp.float32),
        pltpu.VMEM_SHARED((_VP, _D), jnp.float32),
        pltpu.VMEM_SHARED((_VP, _D), jnp.float32),
        pltpu.VMEM_SHARED((_VP, 16), jnp.float32),
        pltpu.SemaphoreType.DMA,
        pltpu.SemaphoreType.DMA,
        pltpu.SemaphoreType.DMA,
        pltpu.SemaphoreType.DMA,
        pltpu.SemaphoreType.DMA,
    ],
    compiler_params=pltpu.CompilerParams(
        use_tc_tiling_on_sc=False, needs_layout_passes=False,
        disable_bounds_checks=True),
)(_phase1_body)


def _finalize_body(a_ref, cnt_ref, loss_ref, nrep_ref):
    a = a_ref[0] + a_ref[1]
    cnt = cnt_ref[0] + cnt_ref[1]
    c = cnt.reshape(_VP // 8, 8, 16)[:, :, 0:1].reshape(_VP, 1)
    cm = jnp.maximum(c, 1.0)
    lane = lax.broadcasted_iota(jnp.int32, (_VP, 2 * _D), 1)
    contrib = jnp.where(lane >= _D, a, -(a * a) / cm)
    ss_sum = jnp.sum(contrib, axis=1, keepdims=True)
    var_mean = ss_sum / (jnp.maximum(c - 1.0, 1.0) * _D)
    repeated = c >= 2.0
    nrep = jnp.sum(repeated.astype(jnp.float32))
    total = jnp.sum(jnp.where(repeated, var_mean, 0.0))
    avg = total / jnp.maximum(nrep, 1.0)
    loss = jnp.maximum(1.0 - avg, 0.0)
    loss = jnp.where(nrep > 0.0, loss, 0.0)
    loss_ref[0, 0] = loss
    nrep_ref[0, 0] = nrep.astype(jnp.int32)


_finalize = pl.pallas_call(
    _finalize_body,
    out_shape=(
        jax.ShapeDtypeStruct((1, 1), jnp.float32),
        jax.ShapeDtypeStruct((1, 1), jnp.int32),
    ),
    out_specs=(
        pl.BlockSpec(memory_space=pltpu.SMEM),
        pl.BlockSpec(memory_space=pltpu.SMEM),
    ),
)


@jax.jit
def kernel(semantic_state, token_ids):
    x5 = jnp.transpose(
        semantic_state.reshape(_B, _T // 128, 128, _D // 8, 8),
        (0, 3, 1, 4, 2))
    x6 = x5.reshape(_B, _D // 8, _T // 128, 8 * 128)
    tok = jnp.transpose(
        token_ids.astype(jnp.int32).reshape(_B, _NW, _CH), (1, 0, 2))
    pa, pcnt = _phase1(x6, tok)
    loss, nrep = _finalize(pa, pcnt)
    return loss[0, 0], nrep[0, 0]
